# Initial kernel scaffold; baseline (speedup 1.0000x reference)
import jax
import jax.numpy as jnp
from jax.experimental import pallas as pl


def kernel(idx, E):
    raise NotImplementedError("write your pallas kernel here")


# speedup vs baseline: 1.8730x; 1.8730x over previous
import functools

import jax
import jax.numpy as jnp
from jax import lax
from jax.experimental import pallas as pl
from jax.experimental.pallas import tpu as pltpu
from jax.experimental.pallas import tpu_sc as plsc

D_MODEL = 64
NUM_CORES = 2
NUM_SUBCORES = 16
NW = NUM_CORES * NUM_SUBCORES

CHUNK = 128
NBUF = 8


@functools.lru_cache(maxsize=None)
def _build(B):
    assert B % (NW * CHUNK * NBUF) == 0
    chunks_per_w = B // (NW * CHUNK)
    groups = chunks_per_w // NBUF
    mesh = plsc.VectorSubcoreMesh(core_axis_name="c", subcore_axis_name="s")

    @functools.partial(
        pl.kernel,
        mesh=mesh,
        compiler_params=pltpu.CompilerParams(use_tc_tiling_on_sc=False),
        out_type=jax.ShapeDtypeStruct((B, D_MODEL), jnp.float32),
        scratch_types=[
            pltpu.VMEM((chunks_per_w, CHUNK), jnp.int32),
            pltpu.VMEM((NBUF, CHUNK, D_MODEL), jnp.float32),
            pltpu.SemaphoreType.DMA((NBUF,)),
            pltpu.SemaphoreType.DMA((NBUF,)),
        ],
    )
    def emb(idx_hbm, table_hbm, out_hbm, idx_v, rows_v, gsem, osem):
        wid = lax.axis_index("s") * NUM_CORES + lax.axis_index("c")
        chunk0 = wid * chunks_per_w

        pltpu.sync_copy(idx_hbm.at[pl.ds(chunk0, chunks_per_w)], idx_v)

        def gather_start(c, b):
            pltpu.async_copy(table_hbm.at[idx_v.at[c]], rows_v.at[b],
                             gsem.at[b])

        def gather_wait(c, b):
            pltpu.make_async_copy(table_hbm.at[idx_v.at[c]], rows_v.at[b],
                                  gsem.at[b]).wait()

        def out_start(c, b):
            pltpu.async_copy(rows_v.at[b],
                             out_hbm.at[pl.ds((chunk0 + c) * CHUNK, CHUNK)],
                             osem.at[b])

        def out_wait(c, b):
            pltpu.make_async_copy(rows_v.at[b],
                                  out_hbm.at[pl.ds((chunk0 + c) * CHUNK,
                                                   CHUNK)],
                                  osem.at[b]).wait()

        for b in range(NBUF):
            gather_start(b, b)

        def body(g, carry):
            cbase = g * NBUF
            for b in range(NBUF):
                gather_wait(cbase + b, b)
                out_start(cbase + b, b)

            @pl.when(g < groups - 1)
            def _next():
                for b in range(NBUF):
                    out_wait(cbase + b, b)
                    gather_start(cbase + NBUF + b, b)

            return carry

        lax.fori_loop(0, groups, body, 0, unroll=False)

        cbase = (groups - 1) * NBUF
        for b in range(NBUF):
            out_wait(cbase + b, b)

    return emb


def kernel(idx, E):
    nb, nt = idx.shape
    B = nb * nt
    idx32 = idx.astype(jnp.int32).reshape(B // CHUNK, CHUNK)
    out = _build(B)(idx32, E)
    return out.reshape(nb, nt, D_MODEL)


# speedup vs baseline: 1.8731x; 1.0001x over previous
import functools

import jax
import jax.numpy as jnp
from jax import lax
from jax.experimental import pallas as pl
from jax.experimental.pallas import tpu as pltpu
from jax.experimental.pallas import tpu_sc as plsc

D_MODEL = 64
NUM_CORES = 2
NUM_SUBCORES = 16
NW = NUM_CORES * NUM_SUBCORES

CHUNK = 128
NBUF = 10
LA = 5


@functools.lru_cache(maxsize=None)
def _build(B):
    assert B % (NW * CHUNK * NBUF) == 0
    chunks_per_w = B // (NW * CHUNK)
    groups = chunks_per_w // NBUF
    mesh = plsc.VectorSubcoreMesh(core_axis_name="c", subcore_axis_name="s")

    @functools.partial(
        pl.kernel,
        mesh=mesh,
        compiler_params=pltpu.CompilerParams(use_tc_tiling_on_sc=False),
        out_type=jax.ShapeDtypeStruct((B, D_MODEL), jnp.float32),
        scratch_types=[
            pltpu.VMEM((chunks_per_w, CHUNK), jnp.int32),
            pltpu.VMEM((NBUF, CHUNK, D_MODEL), jnp.float32),
            pltpu.SemaphoreType.DMA((NBUF,)),
            pltpu.SemaphoreType.DMA((NBUF,)),
        ],
    )
    def emb(idx_hbm, table_hbm, out_hbm, idx_v, rows_v, gsem, osem):
        wid = lax.axis_index("s") * NUM_CORES + lax.axis_index("c")
        chunk0 = wid * chunks_per_w

        pltpu.sync_copy(idx_hbm.at[pl.ds(chunk0, chunks_per_w)], idx_v)

        def gather_start(c, b):
            pltpu.async_copy(table_hbm.at[idx_v.at[c]], rows_v.at[b],
                             gsem.at[b])

        def gather_wait(c, b):
            pltpu.make_async_copy(table_hbm.at[idx_v.at[c]], rows_v.at[b],
                                  gsem.at[b]).wait()

        def out_start(c, b):
            pltpu.async_copy(rows_v.at[b],
                             out_hbm.at[pl.ds((chunk0 + c) * CHUNK, CHUNK)],
                             osem.at[b])

        def out_wait(c, b):
            pltpu.make_async_copy(rows_v.at[b],
                                  out_hbm.at[pl.ds((chunk0 + c) * CHUNK,
                                                   CHUNK)],
                                  osem.at[b]).wait()

        nchunk = chunks_per_w

        for b in range(LA):
            gather_start(b, b)

        def body(g, carry):
            cbase = g * NBUF
            for b in range(NBUF):
                c = cbase + b
                gather_wait(c, b)
                out_start(c, b)
                c2 = c + LA
                b2 = (b + LA) % NBUF

                @pl.when(c2 < nchunk)
                def _issue():
                    @pl.when(c2 >= NBUF)
                    def _free():
                        out_wait(c2 - NBUF, b2)

                    gather_start(c2, b2)

            return carry

        lax.fori_loop(0, groups, body, 0, unroll=False)

        cbase = (groups - 1) * NBUF
        for b in range(NBUF):
            out_wait(cbase + b, b)

    return emb


def kernel(idx, E):
    nb, nt = idx.shape
    B = nb * nt
    idx32 = idx.astype(jnp.int32).reshape(B // CHUNK, CHUNK)
    out = _build(B)(idx32, E)
    return out.reshape(nb, nt, D_MODEL)


# speedup vs baseline: 1.8742x; 1.0006x over previous
import functools

import jax
import jax.numpy as jnp
from jax import lax
from jax.experimental import pallas as pl
from jax.experimental.pallas import tpu as pltpu
from jax.experimental.pallas import tpu_sc as plsc

D_MODEL = 64
NUM_CORES = 2
NUM_SUBCORES = 16
NW = NUM_CORES * NUM_SUBCORES

T_DIM = 50
CHUNK = 100
NBUF = 8
LA = 4


@functools.lru_cache(maxsize=None)
def _build(NB):
    B = NB * T_DIM
    assert B % (NW * CHUNK * NBUF) == 0
    chunks_per_w = B // (NW * CHUNK)
    groups = chunks_per_w // NBUF
    mesh = plsc.VectorSubcoreMesh(core_axis_name="c", subcore_axis_name="s")

    @functools.partial(
        pl.kernel,
        mesh=mesh,
        compiler_params=pltpu.CompilerParams(use_tc_tiling_on_sc=False),
        out_type=jax.ShapeDtypeStruct((NB, T_DIM, D_MODEL), jnp.float32),
        scratch_types=[
            pltpu.VMEM((chunks_per_w, CHUNK), jnp.int32),
            pltpu.VMEM((NBUF, CHUNK, D_MODEL), jnp.float32),
            pltpu.SemaphoreType.DMA((NBUF,)),
            pltpu.SemaphoreType.DMA((NBUF,)),
        ],
    )
    def emb(idx_hbm, table_hbm, out_hbm, idx_v, rows_v, gsem, osem):
        wid = lax.axis_index("s") * NUM_CORES + lax.axis_index("c")
        chunk0 = wid * chunks_per_w

        pltpu.sync_copy(idx_hbm.at[pl.ds(chunk0, chunks_per_w)], idx_v)

        def gather_start(c, b):
            pltpu.async_copy(table_hbm.at[idx_v.at[c]], rows_v.at[b],
                             gsem.at[b])

        def gather_wait(c, b):
            pltpu.make_async_copy(table_hbm.at[idx_v.at[c]], rows_v.at[b],
                                  gsem.at[b]).wait()

        def out_start(c, b):
            bb = (chunk0 + c) * 2
            pltpu.async_copy(rows_v.at[b, pl.ds(0, T_DIM)], out_hbm.at[bb],
                             osem.at[b])
            pltpu.async_copy(rows_v.at[b, pl.ds(T_DIM, T_DIM)],
                             out_hbm.at[bb + 1], osem.at[b])

        def out_wait(c, b):
            bb = (chunk0 + c) * 2
            pltpu.make_async_copy(rows_v.at[b, pl.ds(0, T_DIM)],
                                  out_hbm.at[bb], osem.at[b]).wait()
            pltpu.make_async_copy(rows_v.at[b, pl.ds(T_DIM, T_DIM)],
                                  out_hbm.at[bb + 1], osem.at[b]).wait()

        nchunk = chunks_per_w

        for b in range(LA):
            gather_start(b, b)

        def body(g, carry):
            cbase = g * NBUF
            for b in range(NBUF):
                c = cbase + b
                gather_wait(c, b)
                out_start(c, b)
                c2 = c + LA
                b2 = (b + LA) % NBUF

                @pl.when(c2 < nchunk)
                def _issue():
                    @pl.when(c2 >= NBUF)
                    def _free():
                        out_wait(c2 - NBUF, b2)

                    gather_start(c2, b2)

            return carry

        lax.fori_loop(0, groups, body, 0, unroll=False)

        cbase = (groups - 1) * NBUF
        for b in range(NBUF):
            out_wait(cbase + b, b)

    return emb


def kernel(idx, E):
    nb, nt = idx.shape
    B = nb * nt
    idx32 = idx.astype(jnp.int32).reshape(B // CHUNK, CHUNK)
    return _build(nb)(idx32, E)
